# Initial kernel scaffold; baseline (speedup 1.0000x reference)
import jax
import jax.numpy as jnp
from jax.experimental import pallas as pl


def kernel(inputs, weight):
    raise NotImplementedError("write your pallas kernel here")


# speedup vs baseline: 1.0058x; 1.0058x over previous
import functools

import jax
import jax.numpy as jnp
from jax import lax
from jax.experimental import pallas as pl
from jax.experimental.pallas import tpu as pltpu
from jax.experimental.pallas import tpu_sc as plsc

_COMMIT = 0.25
_TAU = 0.5

_TN = 512
_TK = 1024
_TN2 = 256
_TK2 = 1024
_NC = 2
_NS = 16


def _dist_body(xn_ref, wn_ref, x_ref, w_ref, g_ref, ent_ref, idx_ref,
               m_s, s_s, u_s, best_s, bidx_s):
    j = pl.program_id(1)
    nk = pl.num_programs(1)

    @pl.when(j == 0)
    def _init():
        m_s[...] = jnp.full(m_s.shape, -1e30, jnp.float32)
        s_s[...] = jnp.zeros(s_s.shape, jnp.float32)
        u_s[...] = jnp.zeros(u_s.shape, jnp.float32)
        best_s[...] = jnp.full(best_s.shape, -jnp.inf, jnp.float32)
        bidx_s[...] = jnp.zeros(bidx_s.shape, jnp.int32)

    dot = lax.dot_general(
        x_ref[...].astype(jnp.bfloat16), w_ref[...].astype(jnp.bfloat16),
        (((1,), (1,)), ((), ())), preferred_element_type=jnp.float32)
    d = (xn_ref[...] + wn_ref[...]) - 2.0 * dot
    sc = g_ref[...] - d
    tmax = jnp.max(sc, axis=1, keepdims=True)
    col = lax.broadcasted_iota(jnp.int32, sc.shape, 1) + j * _TK
    cand = jnp.min(jnp.where(sc == tmax, col, jnp.int32(2 ** 30)),
                   axis=1, keepdims=True)
    upd = tmax > best_s[...]
    best_s[...] = jnp.where(upd, tmax, best_s[...])
    bidx_s[...] = jnp.where(upd, cand, bidx_s[...])

    l = -2.0 * d
    mt = jnp.max(l, axis=1, keepdims=True)
    m_old = m_s[...]
    m_new = jnp.maximum(m_old, mt)
    scale = jnp.exp(m_old - m_new)
    t = l - m_new
    e = jnp.exp(t)
    s_t = jnp.sum(e, axis=1, keepdims=True)
    u_t = jnp.sum(t * e, axis=1, keepdims=True)
    s_old = s_s[...]
    u_s[...] = (u_s[...] + (m_old - m_new) * s_old) * scale + u_t
    s_s[...] = s_old * scale + s_t
    m_s[...] = m_new

    @pl.when(j == nk - 1)
    def _fin():
        s = s_s[...]
        ent_ref[...] = u_s[...] / s - jnp.log(s)
        idx_ref[...] = bidx_s[...]


def _dist_call(xn, wn, flat, weight, g):
    n, ddim = flat.shape
    k = weight.shape[0]
    grid = (n // _TN, k // _TK)
    return pl.pallas_call(
        _dist_body,
        grid=grid,
        in_specs=[
            pl.BlockSpec((_TN, 1), lambda i, j: (i, 0)),
            pl.BlockSpec((1, _TK), lambda i, j: (0, j)),
            pl.BlockSpec((_TN, ddim), lambda i, j: (i, 0)),
            pl.BlockSpec((_TK, ddim), lambda i, j: (j, 0)),
            pl.BlockSpec((_TN, _TK), lambda i, j: (i, j)),
        ],
        out_specs=[
            pl.BlockSpec((_TN, 1), lambda i, j: (i, 0)),
            pl.BlockSpec((_TN, 1), lambda i, j: (i, 0)),
        ],
        out_shape=[
            jax.ShapeDtypeStruct((n, 1), jnp.float32),
            jax.ShapeDtypeStruct((n, 1), jnp.int32),
        ],
        scratch_shapes=[
            pltpu.VMEM((_TN, 1), jnp.float32),
            pltpu.VMEM((_TN, 1), jnp.float32),
            pltpu.VMEM((_TN, 1), jnp.float32),
            pltpu.VMEM((_TN, 1), jnp.float32),
            pltpu.VMEM((_TN, 1), jnp.int32),
        ],
        compiler_params=pltpu.CompilerParams(
            dimension_semantics=("arbitrary", "arbitrary")),
    )(xn, wn, flat, weight, g)


def _onehot_body(idx_ref, ent_ref, enc_ref, loss_ref, perp_ref,
                 cnt_s, esum_s, pent_s, n_total):
    jj = pl.program_id(0)
    ii = pl.program_id(1)
    nj = pl.num_programs(0)
    ni = pl.num_programs(1)

    idx = idx_ref[...]
    col = lax.broadcasted_iota(jnp.int32, (_TN2, _TK2), 1) + jj * _TK2
    e = (col == idx).astype(jnp.float32)
    enc_ref[...] = e

    @pl.when(ii == 0)
    def _zero_cnt():
        cnt_s[...] = jnp.zeros(cnt_s.shape, jnp.float32)

    cnt_s[...] += jnp.sum(e, axis=0, keepdims=True)

    @pl.when(jnp.logical_and(jj == 0, ii == 0))
    def _zero_scalars():
        esum_s[...] = jnp.zeros(esum_s.shape, jnp.float32)
        pent_s[...] = jnp.zeros(pent_s.shape, jnp.float32)

    @pl.when(jj == 0)
    def _acc_ent():
        esum_s[...] += jnp.sum(ent_ref[...])

    @pl.when(ii == ni - 1)
    def _acc_perp():
        avg = cnt_s[...] / float(n_total)
        pent_s[...] += jnp.sum(avg * jnp.log(avg + 1e-10))

        @pl.when(jj == nj - 1)
        def _fin():
            loss_ref[...] = _COMMIT * (esum_s[...] / float(n_total))
            perp_ref[...] = jnp.exp(-pent_s[...])


def _onehot_call(idx, ent):
    n = idx.shape[0]
    k = 8192
    grid = (k // _TK2, n // _TN2)
    body = functools.partial(_onehot_body, n_total=n)
    return pl.pallas_call(
        body,
        grid=grid,
        in_specs=[
            pl.BlockSpec((_TN2, 1), lambda jj, ii: (ii, 0)),
            pl.BlockSpec((_TN2, 1), lambda jj, ii: (ii, 0)),
        ],
        out_specs=[
            pl.BlockSpec((_TN2, _TK2), lambda jj, ii: (ii, jj)),
            pl.BlockSpec((1, 1), lambda jj, ii: (0, 0)),
            pl.BlockSpec((1, 1), lambda jj, ii: (0, 0)),
        ],
        out_shape=[
            jax.ShapeDtypeStruct((n, k), jnp.float32),
            jax.ShapeDtypeStruct((1, 1), jnp.float32),
            jax.ShapeDtypeStruct((1, 1), jnp.float32),
        ],
        scratch_shapes=[
            pltpu.VMEM((1, _TK2), jnp.float32),
            pltpu.VMEM((1, 1), jnp.float32),
            pltpu.VMEM((1, 1), jnp.float32),
        ],
        compiler_params=pltpu.CompilerParams(
            dimension_semantics=("arbitrary", "arbitrary")),
    )(idx, ent)


def _sc_gather_build(n, ddim):
    bpw = n // (_NC * _NS)
    mesh = plsc.VectorSubcoreMesh(core_axis_name="c", subcore_axis_name="s")

    @functools.partial(
        pl.kernel,
        out_type=jax.ShapeDtypeStruct((n, ddim), jnp.float32),
        mesh=mesh,
        scratch_types=[
            pltpu.VMEM((bpw,), jnp.int32),
            pltpu.VMEM((bpw, ddim), jnp.float32),
            pltpu.SemaphoreType.DMA,
        ],
    )
    def _sc_gather(w_hbm, idx_hbm, out_hbm, idx_v, rows_v, sem):
        wid = lax.axis_index("s") * _NC + lax.axis_index("c")
        base = wid * bpw
        pltpu.sync_copy(idx_hbm.at[pl.ds(base, bpw)], idx_v)
        pltpu.async_copy(w_hbm.at[idx_v], rows_v, sem).wait()
        pltpu.sync_copy(rows_v, out_hbm.at[pl.ds(base, bpw)])

    return _sc_gather


def kernel(inputs, weight):
    x = jnp.transpose(inputs, (0, 2, 3, 1))
    b, h, w, dd = x.shape
    flat = x.reshape(-1, dd)
    n = flat.shape[0]
    k = weight.shape[0]

    xn = jnp.sum(flat ** 2, axis=1, keepdims=True)
    wn = jnp.sum(weight ** 2, axis=1)[None, :]
    g = jax.random.gumbel(jax.random.key(42), (n, k), jnp.float32)

    ent, idx = _dist_call(xn, wn, flat, weight, g)

    quant = _sc_gather_build(n, dd)(weight, idx.reshape(-1))

    enc, loss11, perp11 = _onehot_call(idx, ent)

    q_out = jnp.transpose(quant.reshape(b, h, w, dd), (0, 3, 1, 2))
    return (loss11[0, 0], q_out, perp11[0, 0], enc)


# speedup vs baseline: 1.0373x; 1.0314x over previous
import functools

import jax
import jax.numpy as jnp
from jax import lax
from jax.experimental import pallas as pl
from jax.experimental.pallas import tpu as pltpu
from jax.experimental.pallas import tpu_sc as plsc

_COMMIT = 0.25
_TAU = 0.5

_TN = 512
_TK = 1024
_TN2 = 256
_TK2 = 1024
_NC = 2
_NS = 16


_KS1 = 42
_KS2 = 0x1BD11BF0
_TINY = 1.1754943508222875e-38


def _tf_rounds(x0, x1, rots):
    for r in rots:
        x0 = x0 + x1
        x1 = lax.shift_left(x1, jnp.uint32(r)) | lax.shift_right_logical(
            x1, jnp.uint32(32 - r))
        x1 = x1 ^ x0
    return x0, x1


def _gumbel_tile(i, j, k_total, shape):
    ra = jnp.uint32(13), jnp.uint32(15), jnp.uint32(26), jnp.uint32(6)
    rb = jnp.uint32(17), jnp.uint32(29), jnp.uint32(16), jnp.uint32(24)
    base = (i * (shape[0] * k_total) + j * shape[1] + _KS1).astype(jnp.uint32)
    rr = lax.broadcasted_iota(jnp.uint32, shape, 0)
    cc = lax.broadcasted_iota(jnp.uint32, shape, 1)
    x1 = lax.shift_left(rr, jnp.uint32(13)) + cc + base
    x0 = jnp.zeros(shape, jnp.uint32)
    x0, x1 = _tf_rounds(x0, x1, ra)
    x0, x1 = x0 + jnp.uint32(_KS1), x1 + jnp.uint32(_KS2 + 1)
    x0, x1 = _tf_rounds(x0, x1, rb)
    x0, x1 = x0 + jnp.uint32(_KS2), x1 + jnp.uint32(2)
    x0, x1 = _tf_rounds(x0, x1, ra)
    x0, x1 = x0, x1 + jnp.uint32(_KS1 + 3)
    x0, x1 = _tf_rounds(x0, x1, rb)
    x0, x1 = x0 + jnp.uint32(_KS1), x1 + jnp.uint32(_KS2 + 4)
    x0, x1 = _tf_rounds(x0, x1, ra)
    x0, x1 = x0 + jnp.uint32(_KS2), x1 + jnp.uint32(5)
    bits = x0 ^ x1
    fb = lax.shift_right_logical(bits, jnp.uint32(9)) | jnp.uint32(0x3F800000)
    fl = lax.bitcast_convert_type(fb, jnp.float32) - 1.0
    u = jnp.maximum(jnp.float32(_TINY), fl + jnp.float32(_TINY))
    return -jnp.log(-jnp.log(u))


def _dist_body(xn_ref, wn_ref, x_ref, w_ref, ent_ref, idx_ref,
               m_s, s_s, u_s, best_s, bidx_s, k_total):
    i = pl.program_id(0)
    j = pl.program_id(1)
    nk = pl.num_programs(1)

    @pl.when(j == 0)
    def _init():
        m_s[...] = jnp.full(m_s.shape, -1e30, jnp.float32)
        s_s[...] = jnp.zeros(s_s.shape, jnp.float32)
        u_s[...] = jnp.zeros(u_s.shape, jnp.float32)
        best_s[...] = jnp.full(best_s.shape, -jnp.inf, jnp.float32)
        bidx_s[...] = jnp.zeros(bidx_s.shape, jnp.int32)

    dot = lax.dot_general(
        x_ref[...].astype(jnp.bfloat16), w_ref[...].astype(jnp.bfloat16),
        (((1,), (1,)), ((), ())), preferred_element_type=jnp.float32)
    d = (xn_ref[...] + wn_ref[...]) - 2.0 * dot
    g = _gumbel_tile(i, j, k_total, d.shape)
    sc = g - d
    tmax = jnp.max(sc, axis=1, keepdims=True)
    col = lax.broadcasted_iota(jnp.int32, sc.shape, 1) + j * _TK
    cand = jnp.min(jnp.where(sc == tmax, col, jnp.int32(2 ** 30)),
                   axis=1, keepdims=True)
    upd = tmax > best_s[...]
    best_s[...] = jnp.where(upd, tmax, best_s[...])
    bidx_s[...] = jnp.where(upd, cand, bidx_s[...])

    l = -2.0 * d
    mt = jnp.max(l, axis=1, keepdims=True)
    m_old = m_s[...]
    m_new = jnp.maximum(m_old, mt)
    scale = jnp.exp(m_old - m_new)
    t = l - m_new
    e = jnp.exp(t)
    s_t = jnp.sum(e, axis=1, keepdims=True)
    u_t = jnp.sum(t * e, axis=1, keepdims=True)
    s_old = s_s[...]
    u_s[...] = (u_s[...] + (m_old - m_new) * s_old) * scale + u_t
    s_s[...] = s_old * scale + s_t
    m_s[...] = m_new

    @pl.when(j == nk - 1)
    def _fin():
        s = s_s[...]
        ent_ref[...] = u_s[...] / s - jnp.log(s)
        idx_ref[...] = bidx_s[...]


def _dist_call(xn, wn, flat, weight):
    n, ddim = flat.shape
    k = weight.shape[0]
    grid = (n // _TN, k // _TK)
    return pl.pallas_call(
        functools.partial(_dist_body, k_total=k),
        grid=grid,
        in_specs=[
            pl.BlockSpec((_TN, 1), lambda i, j: (i, 0)),
            pl.BlockSpec((1, _TK), lambda i, j: (0, j)),
            pl.BlockSpec((_TN, ddim), lambda i, j: (i, 0)),
            pl.BlockSpec((_TK, ddim), lambda i, j: (j, 0)),
        ],
        out_specs=[
            pl.BlockSpec((_TN, 1), lambda i, j: (i, 0)),
            pl.BlockSpec((_TN, 1), lambda i, j: (i, 0)),
        ],
        out_shape=[
            jax.ShapeDtypeStruct((n, 1), jnp.float32),
            jax.ShapeDtypeStruct((n, 1), jnp.int32),
        ],
        scratch_shapes=[
            pltpu.VMEM((_TN, 1), jnp.float32),
            pltpu.VMEM((_TN, 1), jnp.float32),
            pltpu.VMEM((_TN, 1), jnp.float32),
            pltpu.VMEM((_TN, 1), jnp.float32),
            pltpu.VMEM((_TN, 1), jnp.int32),
        ],
        compiler_params=pltpu.CompilerParams(
            dimension_semantics=("arbitrary", "arbitrary")),
    )(xn, wn, flat, weight)


def _onehot_body(idx_ref, ent_ref, enc_ref, loss_ref, perp_ref,
                 cnt_s, esum_s, pent_s, n_total):
    jj = pl.program_id(0)
    ii = pl.program_id(1)
    nj = pl.num_programs(0)
    ni = pl.num_programs(1)

    idx = idx_ref[...]
    col = lax.broadcasted_iota(jnp.int32, (_TN2, _TK2), 1) + jj * _TK2
    e = (col == idx).astype(jnp.float32)
    enc_ref[...] = e

    @pl.when(ii == 0)
    def _zero_cnt():
        cnt_s[...] = jnp.zeros(cnt_s.shape, jnp.float32)

    cnt_s[...] += jnp.sum(e, axis=0, keepdims=True)

    @pl.when(jnp.logical_and(jj == 0, ii == 0))
    def _zero_scalars():
        esum_s[...] = jnp.zeros(esum_s.shape, jnp.float32)
        pent_s[...] = jnp.zeros(pent_s.shape, jnp.float32)

    @pl.when(jj == 0)
    def _acc_ent():
        esum_s[...] += jnp.sum(ent_ref[...])

    @pl.when(ii == ni - 1)
    def _acc_perp():
        avg = cnt_s[...] / float(n_total)
        pent_s[...] += jnp.sum(avg * jnp.log(avg + 1e-10))

        @pl.when(jj == nj - 1)
        def _fin():
            loss_ref[...] = _COMMIT * (esum_s[...] / float(n_total))
            perp_ref[...] = jnp.exp(-pent_s[...])


def _onehot_call(idx, ent):
    n = idx.shape[0]
    k = 8192
    grid = (k // _TK2, n // _TN2)
    body = functools.partial(_onehot_body, n_total=n)
    return pl.pallas_call(
        body,
        grid=grid,
        in_specs=[
            pl.BlockSpec((_TN2, 1), lambda jj, ii: (ii, 0)),
            pl.BlockSpec((_TN2, 1), lambda jj, ii: (ii, 0)),
        ],
        out_specs=[
            pl.BlockSpec((_TN2, _TK2), lambda jj, ii: (ii, jj)),
            pl.BlockSpec((1, 1), lambda jj, ii: (0, 0)),
            pl.BlockSpec((1, 1), lambda jj, ii: (0, 0)),
        ],
        out_shape=[
            jax.ShapeDtypeStruct((n, k), jnp.float32),
            jax.ShapeDtypeStruct((1, 1), jnp.float32),
            jax.ShapeDtypeStruct((1, 1), jnp.float32),
        ],
        scratch_shapes=[
            pltpu.VMEM((1, _TK2), jnp.float32),
            pltpu.VMEM((1, 1), jnp.float32),
            pltpu.VMEM((1, 1), jnp.float32),
        ],
        compiler_params=pltpu.CompilerParams(
            dimension_semantics=("arbitrary", "arbitrary")),
    )(idx, ent)


def _sc_gather_build(n, ddim):
    bpw = n // (_NC * _NS)
    mesh = plsc.VectorSubcoreMesh(core_axis_name="c", subcore_axis_name="s")

    @functools.partial(
        pl.kernel,
        out_type=jax.ShapeDtypeStruct((n, ddim), jnp.float32),
        mesh=mesh,
        scratch_types=[
            pltpu.VMEM((bpw,), jnp.int32),
            pltpu.VMEM((bpw, ddim), jnp.float32),
            pltpu.SemaphoreType.DMA,
        ],
    )
    def _sc_gather(w_hbm, idx_hbm, out_hbm, idx_v, rows_v, sem):
        wid = lax.axis_index("s") * _NC + lax.axis_index("c")
        base = wid * bpw
        pltpu.sync_copy(idx_hbm.at[pl.ds(base, bpw)], idx_v)
        pltpu.async_copy(w_hbm.at[idx_v], rows_v, sem).wait()
        pltpu.sync_copy(rows_v, out_hbm.at[pl.ds(base, bpw)])

    return _sc_gather


def kernel(inputs, weight):
    x = jnp.transpose(inputs, (0, 2, 3, 1))
    b, h, w, dd = x.shape
    flat = x.reshape(-1, dd)
    n = flat.shape[0]
    k = weight.shape[0]

    xn = jnp.sum(flat ** 2, axis=1, keepdims=True)
    wn = jnp.sum(weight ** 2, axis=1)[None, :]

    ent, idx = _dist_call(xn, wn, flat, weight)

    quant = _sc_gather_build(n, dd)(weight, idx.reshape(-1))

    enc, loss11, perp11 = _onehot_call(idx, ent)

    q_out = jnp.transpose(quant.reshape(b, h, w, dd), (0, 3, 1, 2))
    return (loss11[0, 0], q_out, perp11[0, 0], enc)
